# baseline (speedup 1.0000x reference)
import functools

import jax
import jax.numpy as jnp
from jax import lax
from jax.experimental import pallas as pl
from jax.experimental.pallas import tpu as pltpu
from jax.experimental.pallas import tpu_sc as plsc

VOC = 100000
EMB = 64
SEQ = 20
HID = 128
B = 1024
EP = 128

BV = 4096


def _sc_gather(table_pad, idx_flat):
    info = plsc.get_sparse_core_info()
    nc, ns = info.num_cores, info.num_subcores
    nw = nc * ns
    n = idx_flat.shape[0]
    b_per_w = n // nw

    mesh = plsc.VectorSubcoreMesh(core_axis_name="c", subcore_axis_name="s")

    @functools.partial(
        pl.kernel,
        mesh=mesh,
        compiler_params=pltpu.CompilerParams(use_tc_tiling_on_sc=False),
        out_type=jax.ShapeDtypeStruct((n, EP), jnp.bfloat16),
        scratch_types=[
            pltpu.VMEM((b_per_w,), jnp.int32),
            pltpu.VMEM((b_per_w, EP), jnp.bfloat16),
            pltpu.SemaphoreType.DMA,
        ],
    )
    def gather_k(table_hbm, idx_hbm, out_hbm, idx_v, rows_v, sem):
        wid = lax.axis_index("s") * nc + lax.axis_index("c")
        base = wid * b_per_w
        pltpu.sync_copy(idx_hbm.at[pl.ds(base, b_per_w)], idx_v)
        pltpu.async_copy(table_hbm.at[idx_v], rows_v, sem).wait()
        pltpu.sync_copy(rows_v, out_hbm.at[pl.ds(base, b_per_w)])

    return gather_k(table_pad, idx_flat)


BT = 16384


def _tp_body(tt_ref, out_ref):
    t = jnp.transpose(tt_ref[...]).astype(jnp.bfloat16)
    out_ref[...] = jnp.concatenate(
        [t, jnp.zeros((t.shape[0], EP - EMB), jnp.bfloat16)], axis=1
    )


def _transpose_pad(tableT):
    nblk = pl.cdiv(VOC, BT)
    return pl.pallas_call(
        _tp_body,
        grid=(nblk,),
        in_specs=[pl.BlockSpec((EMB, BT), lambda j: (0, j))],
        out_specs=pl.BlockSpec((BT, EP), lambda j: (j, 0)),
        out_shape=jax.ShapeDtypeStruct((VOC, EP), jnp.bfloat16),
    )(tableT)


def _mlp_t_body(x_ref, w1_ref, b1_ref, w2t_ref, b2_ref, out_ref, ht_ref):
    @pl.when(pl.program_id(0) == 0)
    def _():
        ht = lax.dot_general(
            w1_ref[0], x_ref[0], (((0,), (1,)), ((), ())),
            preferred_element_type=jnp.float32,
        )
        for s in range(1, SEQ):
            ht += lax.dot_general(
                w1_ref[s], x_ref[s], (((0,), (1,)), ((), ())),
                preferred_element_type=jnp.float32,
            )
        ht_ref[...] = jnp.tanh(ht + b1_ref[...])

    ones_row = jnp.ones((1, B), dtype=jnp.float32)
    bias = lax.dot_general(
        b2_ref[...], ones_row, (((0,), (0,)), ((), ())),
        preferred_element_type=jnp.float32,
    )
    out_ref[...] = (
        jnp.dot(w2t_ref[...], ht_ref[...], preferred_element_type=jnp.float32)
        + bias
    )


def _mlp_t(x3, W1p, b1_col, W2T, b2_row):
    nblk = pl.cdiv(VOC, BV)
    return pl.pallas_call(
        _mlp_t_body,
        grid=(nblk,),
        in_specs=[
            pl.BlockSpec((SEQ, B, EP), lambda j: (0, 0, 0)),
            pl.BlockSpec((SEQ, EP, HID), lambda j: (0, 0, 0)),
            pl.BlockSpec((HID, 1), lambda j: (0, 0)),
            pl.BlockSpec((BV, HID), lambda j: (j, 0)),
            pl.BlockSpec((1, BV), lambda j: (0, j)),
        ],
        out_specs=pl.BlockSpec((BV, B), lambda j: (j, 0)),
        out_shape=jax.ShapeDtypeStruct((VOC, B), jnp.float32),
        scratch_shapes=[pltpu.VMEM((HID, B), jnp.float32)],
    )(x3, W1p, b1_col, W2T, b2_row)


def kernel(input_x, table, W1, b1, W2, b2):
    idx_flat = input_x.T.reshape(-1)
    table_pad = _transpose_pad(table.T)
    emb = _sc_gather(table_pad, idx_flat)
    x3 = emb.reshape(SEQ, B, EP)
    W1p = jnp.pad(W1.reshape(SEQ, EMB, HID), ((0, 0), (0, EP - EMB), (0, 0))).astype(jnp.bfloat16)
    out_t = _mlp_t(x3, W1p, b1.reshape(HID, 1), W2.T, b2.reshape(1, VOC))
    return out_t.T


# speedup vs baseline: 1.5292x; 1.5292x over previous
import functools

import jax
import jax.numpy as jnp
from jax import lax
from jax.experimental import pallas as pl
from jax.experimental.pallas import tpu as pltpu
from jax.experimental.pallas import tpu_sc as plsc

VOC = 100000
EMB = 64
SEQ = 20
HID = 128
B = 1024
EP = 128

BV = 4096


def _sc_gather(table_pad, idx_flat):
    info = plsc.get_sparse_core_info()
    nc, ns = info.num_cores, info.num_subcores
    nw = nc * ns
    n = idx_flat.shape[0]
    b_per_w = n // nw

    mesh = plsc.VectorSubcoreMesh(core_axis_name="c", subcore_axis_name="s")

    @functools.partial(
        pl.kernel,
        mesh=mesh,
        compiler_params=pltpu.CompilerParams(use_tc_tiling_on_sc=False),
        out_type=jax.ShapeDtypeStruct((n, EP), jnp.float32),
        scratch_types=[
            pltpu.VMEM((b_per_w,), jnp.int32),
            pltpu.VMEM((b_per_w, EP), jnp.float32),
            pltpu.SemaphoreType.DMA,
        ],
    )
    def gather_k(table_hbm, idx_hbm, out_hbm, idx_v, rows_v, sem):
        wid = lax.axis_index("s") * nc + lax.axis_index("c")
        base = wid * b_per_w
        pltpu.sync_copy(idx_hbm.at[pl.ds(base, b_per_w)], idx_v)
        pltpu.async_copy(table_hbm.at[idx_v], rows_v, sem).wait()
        pltpu.sync_copy(rows_v, out_hbm.at[pl.ds(base, b_per_w)])

    return gather_k(table_pad, idx_flat)


BT = 16384


def _tp_body(tt_ref, out_ref):
    t = jnp.transpose(tt_ref[...])
    out_ref[...] = jnp.concatenate(
        [t, jnp.zeros((t.shape[0], EP - EMB), jnp.float32)], axis=1
    )


def _transpose_pad(tableT):
    nblk = pl.cdiv(VOC, BT)
    return pl.pallas_call(
        _tp_body,
        grid=(nblk,),
        in_specs=[pl.BlockSpec((EMB, BT), lambda j: (0, j))],
        out_specs=pl.BlockSpec((BT, EP), lambda j: (j, 0)),
        out_shape=jax.ShapeDtypeStruct((VOC, EP), jnp.float32),
    )(tableT)


def _mlp_t_body(x_ref, w1_ref, b1_ref, w2t_ref, b2_ref, out_ref, ht_ref):
    @pl.when(pl.program_id(0) == 0)
    def _():
        ht = lax.dot_general(
            w1_ref[0], x_ref[0], (((0,), (1,)), ((), ())),
            preferred_element_type=jnp.float32,
        )
        for s in range(1, SEQ):
            ht += lax.dot_general(
                w1_ref[s], x_ref[s], (((0,), (1,)), ((), ())),
                preferred_element_type=jnp.float32,
            )
        ht_ref[...] = jnp.tanh(ht + b1_ref[...])

    ones_row = jnp.ones((1, B), dtype=jnp.float32)
    bias = lax.dot_general(
        b2_ref[...], ones_row, (((0,), (0,)), ((), ())),
        preferred_element_type=jnp.float32,
    )
    out_ref[...] = (
        jnp.dot(w2t_ref[...], ht_ref[...], preferred_element_type=jnp.float32)
        + bias
    )


def _mlp_t(x3, W1p, b1_col, W2T, b2_row):
    nblk = pl.cdiv(VOC, BV)
    return pl.pallas_call(
        _mlp_t_body,
        grid=(nblk,),
        in_specs=[
            pl.BlockSpec((SEQ, B, EP), lambda j: (0, 0, 0)),
            pl.BlockSpec((SEQ, EP, HID), lambda j: (0, 0, 0)),
            pl.BlockSpec((HID, 1), lambda j: (0, 0)),
            pl.BlockSpec((BV, HID), lambda j: (j, 0)),
            pl.BlockSpec((1, BV), lambda j: (0, j)),
        ],
        out_specs=pl.BlockSpec((BV, B), lambda j: (j, 0)),
        out_shape=jax.ShapeDtypeStruct((VOC, B), jnp.float32),
        scratch_shapes=[pltpu.VMEM((HID, B), jnp.float32)],
    )(x3, W1p, b1_col, W2T, b2_row)


def kernel(input_x, table, W1, b1, W2, b2):
    idx_flat = input_x.T.reshape(-1)
    table_pad = _transpose_pad(table.T)
    emb = _sc_gather(table_pad, idx_flat)
    x3 = emb.reshape(SEQ, B, EP)
    W1p = jnp.pad(W1.reshape(SEQ, EMB, HID), ((0, 0), (0, EP - EMB), (0, 0)))
    out_t = _mlp_t(x3, W1p, b1.reshape(HID, 1), W2.T, b2.reshape(1, VOC))
    return out_t.T


# speedup vs baseline: 1.5292x; 1.0001x over previous
import functools

import jax
import jax.numpy as jnp
from jax import lax
from jax.experimental import pallas as pl
from jax.experimental.pallas import tpu as pltpu
from jax.experimental.pallas import tpu_sc as plsc

VOC = 100000
EMB = 64
SEQ = 20
HID = 128
B = 1024
EP = 128

BV = 4096


def _sc_gather(table_pad, idx_flat):
    info = plsc.get_sparse_core_info()
    nc, ns = info.num_cores, info.num_subcores
    nw = nc * ns
    n = idx_flat.shape[0]
    b_per_w = n // nw

    mesh = plsc.VectorSubcoreMesh(core_axis_name="c", subcore_axis_name="s")

    half = b_per_w // 2

    @functools.partial(
        pl.kernel,
        mesh=mesh,
        compiler_params=pltpu.CompilerParams(use_tc_tiling_on_sc=False),
        out_type=jax.ShapeDtypeStruct((n, EP), jnp.float32),
        scratch_types=[
            pltpu.VMEM((half,), jnp.int32),
            pltpu.VMEM((half,), jnp.int32),
            pltpu.VMEM((half, EP), jnp.float32),
            pltpu.VMEM((half, EP), jnp.float32),
            pltpu.SemaphoreType.DMA,
            pltpu.SemaphoreType.DMA,
            pltpu.SemaphoreType.DMA,
            pltpu.SemaphoreType.DMA,
        ],
    )
    def gather_k(table_hbm, idx_hbm, out_hbm, idx_a, idx_b, rows_a, rows_b,
                 sem_a, sem_b, sem_oa, sem_ob):
        wid = lax.axis_index("s") * nc + lax.axis_index("c")
        base = wid * b_per_w
        pltpu.sync_copy(idx_hbm.at[pl.ds(base, half)], idx_a)
        ga = pltpu.async_copy(table_hbm.at[idx_a], rows_a, sem_a)
        pltpu.sync_copy(idx_hbm.at[pl.ds(base + half, half)], idx_b)
        gb = pltpu.async_copy(table_hbm.at[idx_b], rows_b, sem_b)
        ga.wait()
        oa = pltpu.async_copy(rows_a, out_hbm.at[pl.ds(base, half)], sem_oa)
        gb.wait()
        ob = pltpu.async_copy(rows_b, out_hbm.at[pl.ds(base + half, half)], sem_ob)
        oa.wait()
        ob.wait()

    return gather_k(table_pad, idx_flat)


BT = 16384


def _tp_body(tt_ref, out_ref):
    t = jnp.transpose(tt_ref[...])
    out_ref[...] = jnp.concatenate(
        [t, jnp.zeros((t.shape[0], EP - EMB), jnp.float32)], axis=1
    )


def _transpose_pad(tableT):
    nblk = pl.cdiv(VOC, BT)
    return pl.pallas_call(
        _tp_body,
        grid=(nblk,),
        in_specs=[pl.BlockSpec((EMB, BT), lambda j: (0, j))],
        out_specs=pl.BlockSpec((BT, EP), lambda j: (j, 0)),
        out_shape=jax.ShapeDtypeStruct((VOC, EP), jnp.float32),
    )(tableT)


def _mlp_t_body(x_ref, w1_ref, b1_ref, w2t_ref, b2_ref, out_ref, ht_ref):
    @pl.when(pl.program_id(0) == 0)
    def _():
        ht = lax.dot_general(
            w1_ref[0], x_ref[0], (((0,), (1,)), ((), ())),
            preferred_element_type=jnp.float32,
        )
        for s in range(1, SEQ):
            ht += lax.dot_general(
                w1_ref[s], x_ref[s], (((0,), (1,)), ((), ())),
                preferred_element_type=jnp.float32,
            )
        ht_ref[...] = jnp.tanh(ht + b1_ref[...])

    ones_row = jnp.ones((1, B), dtype=jnp.float32)
    bias = lax.dot_general(
        b2_ref[...], ones_row, (((0,), (0,)), ((), ())),
        preferred_element_type=jnp.float32,
    )
    out_ref[...] = (
        jnp.dot(w2t_ref[...], ht_ref[...], preferred_element_type=jnp.float32)
        + bias
    )


def _mlp_t(x3, W1p, b1_col, W2T, b2_row):
    nblk = pl.cdiv(VOC, BV)
    return pl.pallas_call(
        _mlp_t_body,
        grid=(nblk,),
        in_specs=[
            pl.BlockSpec((SEQ, B, EP), lambda j: (0, 0, 0)),
            pl.BlockSpec((SEQ, EP, HID), lambda j: (0, 0, 0)),
            pl.BlockSpec((HID, 1), lambda j: (0, 0)),
            pl.BlockSpec((BV, HID), lambda j: (j, 0)),
            pl.BlockSpec((1, BV), lambda j: (0, j)),
        ],
        out_specs=pl.BlockSpec((BV, B), lambda j: (j, 0)),
        out_shape=jax.ShapeDtypeStruct((VOC, B), jnp.float32),
        scratch_shapes=[pltpu.VMEM((HID, B), jnp.float32)],
    )(x3, W1p, b1_col, W2T, b2_row)


def kernel(input_x, table, W1, b1, W2, b2):
    idx_flat = input_x.T.reshape(-1)
    table_pad = _transpose_pad(table.T)
    emb = _sc_gather(table_pad, idx_flat)
    x3 = emb.reshape(SEQ, B, EP)
    W1p = jnp.pad(W1.reshape(SEQ, EMB, HID), ((0, 0), (0, EP - EMB), (0, 0)))
    out_t = _mlp_t(x3, W1p, b1.reshape(HID, 1), W2.T, b2.reshape(1, VOC))
    return out_t.T


# speedup vs baseline: 1.5349x; 1.0037x over previous
import functools

import jax
import jax.numpy as jnp
from jax import lax
from jax.experimental import pallas as pl
from jax.experimental.pallas import tpu as pltpu
from jax.experimental.pallas import tpu_sc as plsc

VOC = 100000
EMB = 64
SEQ = 20
HID = 128
B = 1024
EP = 128

BV = 4096


def _sc_gather(table_pad, idx_flat):
    info = plsc.get_sparse_core_info()
    nc, ns = info.num_cores, info.num_subcores
    nw = nc * ns
    n = idx_flat.shape[0]
    b_per_w = n // nw

    mesh = plsc.VectorSubcoreMesh(core_axis_name="c", subcore_axis_name="s")

    half = b_per_w // 2

    @functools.partial(
        pl.kernel,
        mesh=mesh,
        compiler_params=pltpu.CompilerParams(use_tc_tiling_on_sc=False),
        out_type=jax.ShapeDtypeStruct((n, EP), jnp.float32),
        scratch_types=[
            pltpu.VMEM((half,), jnp.int32),
            pltpu.VMEM((half,), jnp.int32),
            pltpu.VMEM((half, EP), jnp.float32),
            pltpu.VMEM((half, EP), jnp.float32),
            pltpu.SemaphoreType.DMA,
            pltpu.SemaphoreType.DMA,
            pltpu.SemaphoreType.DMA,
            pltpu.SemaphoreType.DMA,
        ],
    )
    def gather_k(table_hbm, idx_hbm, out_hbm, idx_a, idx_b, rows_a, rows_b,
                 sem_a, sem_b, sem_oa, sem_ob):
        wid = lax.axis_index("s") * nc + lax.axis_index("c")
        base = wid * b_per_w
        pltpu.sync_copy(idx_hbm.at[pl.ds(base, half)], idx_a)
        ga = pltpu.async_copy(table_hbm.at[idx_a], rows_a, sem_a)
        pltpu.sync_copy(idx_hbm.at[pl.ds(base + half, half)], idx_b)
        gb = pltpu.async_copy(table_hbm.at[idx_b], rows_b, sem_b)
        ga.wait()
        oa = pltpu.async_copy(rows_a, out_hbm.at[pl.ds(base, half)], sem_oa)
        gb.wait()
        ob = pltpu.async_copy(rows_b, out_hbm.at[pl.ds(base + half, half)], sem_ob)
        oa.wait()
        ob.wait()

    return gather_k(table_pad, idx_flat)


BT = 32768


def _tp_body(tt_ref, out_ref):
    t = jnp.transpose(tt_ref[...])
    out_ref[...] = jnp.concatenate(
        [t, jnp.zeros((t.shape[0], EP - EMB), jnp.float32)], axis=1
    )


def _transpose_pad(tableT):
    nblk = pl.cdiv(VOC, BT)
    return pl.pallas_call(
        _tp_body,
        grid=(nblk,),
        in_specs=[pl.BlockSpec((EMB, BT), lambda j: (0, j))],
        out_specs=pl.BlockSpec((BT, EP), lambda j: (j, 0)),
        out_shape=jax.ShapeDtypeStruct((VOC, EP), jnp.float32),
    )(tableT)


def _mlp_t_body(x_ref, w1_ref, b1_ref, w2t_ref, b2_ref, out_ref, ht_ref):
    @pl.when(pl.program_id(0) == 0)
    def _():
        ht = lax.dot_general(
            w1_ref[0], x_ref[0], (((0,), (1,)), ((), ())),
            preferred_element_type=jnp.float32,
        )
        for s in range(1, SEQ):
            ht += lax.dot_general(
                w1_ref[s], x_ref[s], (((0,), (1,)), ((), ())),
                preferred_element_type=jnp.float32,
            )
        ht_ref[...] = jnp.tanh(ht + b1_ref[...])

    ones_row = jnp.ones((1, B), dtype=jnp.float32)
    bias = lax.dot_general(
        b2_ref[...], ones_row, (((0,), (0,)), ((), ())),
        preferred_element_type=jnp.float32,
    )
    out_ref[...] = (
        jnp.dot(w2t_ref[...], ht_ref[...], preferred_element_type=jnp.float32)
        + bias
    )


def _mlp_t(x3, W1p, b1_col, W2T, b2_row):
    nblk = pl.cdiv(VOC, BV)
    return pl.pallas_call(
        _mlp_t_body,
        grid=(nblk,),
        in_specs=[
            pl.BlockSpec((SEQ, B, EP), lambda j: (0, 0, 0)),
            pl.BlockSpec((SEQ, EP, HID), lambda j: (0, 0, 0)),
            pl.BlockSpec((HID, 1), lambda j: (0, 0)),
            pl.BlockSpec((BV, HID), lambda j: (j, 0)),
            pl.BlockSpec((1, BV), lambda j: (0, j)),
        ],
        out_specs=pl.BlockSpec((BV, B), lambda j: (j, 0)),
        out_shape=jax.ShapeDtypeStruct((VOC, B), jnp.float32),
        scratch_shapes=[pltpu.VMEM((HID, B), jnp.float32)],
    )(x3, W1p, b1_col, W2T, b2_row)


def kernel(input_x, table, W1, b1, W2, b2):
    idx_flat = input_x.T.reshape(-1)
    table_pad = _transpose_pad(table.T)
    emb = _sc_gather(table_pad, idx_flat)
    x3 = emb.reshape(SEQ, B, EP)
    W1p = jnp.pad(W1.reshape(SEQ, EMB, HID), ((0, 0), (0, EP - EMB), (0, 0)))
    out_t = _mlp_t(x3, W1p, b1.reshape(HID, 1), W2.T, b2.reshape(1, VOC))
    return out_t.T
